# baseline (speedup 1.0000x reference)
import functools

import jax
import jax.numpy as jnp
from jax import lax
from jax.experimental import pallas as pl
from jax.experimental.pallas import tpu as pltpu
from jax.experimental.pallas import tpu_sc as plsc

N_USERS = 10000
N_ITEMS = 40000
N = N_USERS + N_ITEMS
DIM = 64
HALF = 32
LAYERS = 3
E = 800000
BATCH = 4096

NC = 2
NS = 16
CH = 128
BLK = 2 * CH
NBLK = E // BLK
OUTER = 98
RPT = N // NS
ZCH = 125
PPT = BATCH // NS


def _sc_body(e0, edata, usersb, itemsb,
             e1, e2, e3, scores,
             acc, ebuf, rowsb, sbuf,
             sem0, sem1, sem2, ssem0, ssem1, ssem2,
             isem0, isem1, isem2, isem3):
    c = lax.axis_index("c")
    s = lax.axis_index("s")
    sems = (sem0, sem1, sem2)
    ssems = (ssem0, ssem1, ssem2)
    isems = (isem0, isem1, isem2, isem3)
    zsrc = rowsb.at[0, pl.ds(0, ZCH), :]

    def zfill():
        def zinit(r, carry):
            rowsb[0, r, pl.ds(0, 16)] = jnp.zeros((16,), jnp.float32)
            rowsb[0, r, pl.ds(16, 16)] = jnp.zeros((16,), jnp.float32)
            return carry
        lax.fori_loop(0, ZCH, zinit, 0, unroll=4)

    zfill()
    for k in range(RPT // ZCH):
        pltpu.sync_copy(zsrc, acc.at[pl.ds(s * RPT + k * ZCH, ZCH), :])
    plsc.subcore_barrier()

    def layer(src, dst):
        def idxload(ib, b3):
            jb = s + NS * ib

            @pl.when(jb < NBLK)
            def _():
                pltpu.async_copy(edata.at[c * NBLK + jb],
                                 ebuf.at[b3], isems[b3])

        def stage(ib, b3, b4):
            jb = s + NS * ib
            jdrain = jb - 3 * NS
            bd4 = (b4 + 1) % 4

            @pl.when(jnp.logical_and(jdrain >= 0, jdrain < NBLK))
            def _():
                for u in range(2):
                    pltpu.make_async_copy(
                        rowsb.at[b3, pl.ds(u * CH, CH), :],
                        acc.at[ebuf.at[bd4, 2 + u]], ssems[b3]).wait()
            idxload(ib + 1, bd4)

            @pl.when(jb < NBLK)
            def _():
                pltpu.make_async_copy(edata.at[c * NBLK + jb],
                                      ebuf.at[b4], isems[b4]).wait()
                for u in range(2):
                    pltpu.async_copy(
                        src.at[ebuf.at[b4, u]],
                        rowsb.at[b3, pl.ds(u * CH, CH), :], sems[b3])

        def process(ib, b3, b4):
            jb = s + NS * ib

            @pl.when(jb < NBLK)
            def _():
                for u in range(2):
                    pltpu.make_async_copy(
                        src.at[ebuf.at[b4, u]],
                        rowsb.at[b3, pl.ds(u * CH, CH), :], sems[b3]).wait()

                def mulq(g, carry2):
                    for u in range(2):
                        wv = plsc.bitcast(
                            ebuf[b4, 4 + u, pl.ds(g * 16, 16)],
                            jnp.float32)
                        for t in range(16):
                            wsv = wv.at[
                                jnp.full((16,), t, jnp.int32)].get(
                                mode="promise_in_bounds")
                            e = u * CH + g * 16 + t
                            rowsb[b3, e, pl.ds(0, 16)] = \
                                rowsb[b3, e, pl.ds(0, 16)] * wsv
                            rowsb[b3, e, pl.ds(16, 16)] = \
                                rowsb[b3, e, pl.ds(16, 16)] * wsv
                    return carry2
                lax.fori_loop(0, CH // 16, mulq, 0)
                for u in range(2):
                    pltpu.async_copy(rowsb.at[b3, pl.ds(u * CH, CH), :],
                                     acc.at[ebuf.at[b4, 2 + u]], ssems[b3],
                                     add=True)

        idxload(0, 0)
        stage(0, 0, 0)
        stage(1, 1, 1)

        def outer(t, carry):
            for k in range(12):
                ib = 12 * t + k
                stage(ib + 2, (k + 2) % 3, (k + 2) % 4)
                process(ib, k % 3, k % 4)
            return carry
        lax.fori_loop(0, 17, outer, 0)
        zfill()
        plsc.subcore_barrier()
        for k in range(RPT // ZCH):
            r0 = s * RPT + k * ZCH
            pltpu.sync_copy(acc.at[pl.ds(r0, ZCH), :],
                            dst.at[pl.ds(c * N + r0, ZCH), :])
            pltpu.sync_copy(zsrc, acc.at[pl.ds(r0, ZCH), :])
        plsc.subcore_barrier()

    layer(e0, e1)
    layer(e1, e2)
    layer(e2, e3)

    lanes = lax.iota(jnp.int32, 16)
    for h in range(2):
        base = c * BATCH + (s * 2 + h) * CH
        pltpu.sync_copy(usersb.at[pl.ds(base, CH)], ebuf.at[0, 0])
        pltpu.sync_copy(itemsb.at[pl.ds(base, CH)], ebuf.at[0, 1])
        for a, arr in enumerate((e0, e1, e2, e3)):
            pltpu.sync_copy(arr.at[ebuf.at[0, 0]],
                            rowsb.at[0, pl.ds(0, CH), :], add=(a > 0))
            pltpu.sync_copy(arr.at[ebuf.at[0, 1]],
                            rowsb.at[0, pl.ds(CH, CH), :], add=(a > 0))

        def dot(g, carry):
            res = jnp.zeros((16,), jnp.float32)
            for t in range(16):
                p = g * 16 + t
                prod = (rowsb[0, p, pl.ds(0, 16)]
                        * rowsb[0, CH + p, pl.ds(0, 16)]
                        + rowsb[0, p, pl.ds(16, 16)]
                        * rowsb[0, CH + p, pl.ds(16, 16)])
                val = jnp.sum(prod) * jnp.float32(1.0 / 16.0)
                res = jnp.where(lanes == t, val, res)
            sbuf[pl.ds(g * 16, 16)] = res
            return carry
        lax.fori_loop(0, CH // 16, dot, 0)
        pltpu.sync_copy(sbuf,
                        scores.at[pl.ds(c * BATCH + s * PPT + h * CH, CH)])


_sc_call = functools.partial(
    pl.kernel,
    out_type=[
        jax.ShapeDtypeStruct((NC * N, HALF), jnp.float32),
        jax.ShapeDtypeStruct((NC * N, HALF), jnp.float32),
        jax.ShapeDtypeStruct((NC * N, HALF), jnp.float32),
        jax.ShapeDtypeStruct((NC * BATCH,), jnp.float32),
    ],
    mesh=plsc.VectorSubcoreMesh(core_axis_name="c", subcore_axis_name="s"),
    compiler_params=pltpu.CompilerParams(use_tc_tiling_on_sc=False,
                                         needs_layout_passes=False),
    scratch_types=[
        pltpu.VMEM_SHARED((N, HALF), jnp.float32),
        pltpu.VMEM((4, 6, CH), jnp.int32),
        pltpu.VMEM((3, BLK, HALF), jnp.float32),
        pltpu.VMEM((CH,), jnp.float32),
        pltpu.SemaphoreType.DMA,
        pltpu.SemaphoreType.DMA,
        pltpu.SemaphoreType.DMA,
        pltpu.SemaphoreType.DMA,
        pltpu.SemaphoreType.DMA,
        pltpu.SemaphoreType.DMA,
        pltpu.SemaphoreType.DMA,
        pltpu.SemaphoreType.DMA,
        pltpu.SemaphoreType.DMA,
        pltpu.SemaphoreType.DMA,
    ],
)(_sc_body)


def kernel(users, items, user_emb, item_emb, edge_index, edge_weight):
    row = edge_index[0]
    col = edge_index[1]
    all_emb = jnp.concatenate([user_emb, item_emb], axis=0)
    e0 = all_emb.reshape(N, NC, HALF).transpose(1, 0, 2).reshape(NC * N, HALF)
    rowp = row.reshape(NBLK, 2, CH)
    wp = lax.bitcast_convert_type(edge_weight, jnp.int32).reshape(NBLK, 2, CH)
    cores = []
    for c in range(NC):
        colp = (col + c * N).reshape(NBLK, 2, CH)
        cores.append(jnp.concatenate([colp, rowp, wp], axis=1))
    edata = jnp.concatenate(cores, axis=0)
    usersb = jnp.concatenate([users, users + N])
    itemsb = jnp.concatenate([items + N_USERS, items + N_USERS + N])
    _, _, _, partial = _sc_call(e0, edata, usersb, itemsb)
    return partial[:BATCH] + partial[BATCH:]


# speedup vs baseline: 1.8839x; 1.8839x over previous
import functools

import jax
import jax.numpy as jnp
from jax import lax
from jax.experimental import pallas as pl
from jax.experimental.pallas import tpu as pltpu
from jax.experimental.pallas import tpu_sc as plsc

N_USERS = 10000
N_ITEMS = 40000
N = N_USERS + N_ITEMS
DIM = 64
HALF = 32
LAYERS = 3
E = 800000
BATCH = 4096

NC = 2
NS = 16
CH = 128
BLK = 2 * CH
NBLK = E // BLK
OUTER = 98
RPT = N // NS
ZCH = 125
PPT = BATCH // NS


def _sc_body(e0, edata, usersb, itemsb,
             e1, e2, e3, scores,
             acc, ebuf, rowsb, zbuf, ub, ib2, sbuf,
             sem0, sem1, ssem0, ssem1, isem0, isem1, isem2):
    c = lax.axis_index("c")
    s = lax.axis_index("s")
    sems = (sem0, sem1)
    ssems = (ssem0, ssem1)
    isems = (isem0, isem1, isem2)

    def zinit(r, carry):
        zbuf[r, pl.ds(0, 16)] = jnp.zeros((16,), jnp.float32)
        zbuf[r, pl.ds(16, 16)] = jnp.zeros((16,), jnp.float32)
        return carry
    lax.fori_loop(0, ZCH, zinit, 0, unroll=4)
    for k in range(RPT // ZCH):
        pltpu.sync_copy(zbuf, acc.at[pl.ds(s * RPT + k * ZCH, ZCH), :])
    plsc.subcore_barrier()

    def layer(src, dst):
        def idxload(ib, b3):
            jb = s + NS * ib

            @pl.when(jb < NBLK)
            def _():
                pltpu.async_copy(edata.at[c * NBLK + jb],
                                 ebuf.at[b3], isems[b3])

        def stage(ib, b2, b3):
            jb = s + NS * ib
            jprev = jb - 2 * NS

            @pl.when(jnp.logical_and(jprev >= 0, jprev < NBLK))
            def _():
                for u in range(2):
                    pltpu.make_async_copy(
                        rowsb.at[b2, pl.ds(u * CH, CH), :],
                        acc.at[ebuf.at[b3, 2 + u]], ssems[b2]).wait()
            idxload(ib + 1, (b3 + 1) % 3)

            @pl.when(jb < NBLK)
            def _():
                pltpu.make_async_copy(edata.at[c * NBLK + jb],
                                      ebuf.at[b3], isems[b3]).wait()
                for u in range(2):
                    pltpu.async_copy(
                        src.at[ebuf.at[b3, u]],
                        rowsb.at[b2, pl.ds(u * CH, CH), :], sems[b2])

        def process(ib, b2, b3):
            jb = s + NS * ib

            @pl.when(jb < NBLK)
            def _():
                for u in range(2):
                    pltpu.make_async_copy(
                        src.at[ebuf.at[b3, u]],
                        rowsb.at[b2, pl.ds(u * CH, CH), :], sems[b2]).wait()

                def mulq(gq, carry2):
                    for u in range(2):
                        for gsub in range(2):
                            g = gq * 2 + gsub
                            wv = plsc.bitcast(
                                ebuf[b3, 4 + u, pl.ds(g * 16, 16)],
                                jnp.float32)
                            for t in range(16):
                                wsv = wv.at[
                                    jnp.full((16,), t, jnp.int32)].get(
                                    mode="promise_in_bounds")
                                e = u * CH + g * 16 + t
                                rowsb[b2, e, pl.ds(0, 16)] = \
                                    rowsb[b2, e, pl.ds(0, 16)] * wsv
                                rowsb[b2, e, pl.ds(16, 16)] = \
                                    rowsb[b2, e, pl.ds(16, 16)] * wsv
                    return carry2
                lax.fori_loop(0, 4, mulq, 0)
                for u in range(2):
                    pltpu.async_copy(rowsb.at[b2, pl.ds(u * CH, CH), :],
                                     acc.at[ebuf.at[b3, 2 + u]], ssems[b2],
                                     add=True)

        idxload(0, 0)
        stage(0, 0, 0)

        def outer(t, carry):
            for k in range(6):
                ib = 6 * t + k
                stage(ib + 1, (k + 1) % 2, (k + 1) % 3)
                process(ib, k % 2, k % 3)
            return carry
        lax.fori_loop(0, 33, outer, 0)
        plsc.subcore_barrier()
        pltpu.sync_copy(acc.at[pl.ds(s * RPT, RPT), :],
                        dst.at[pl.ds(c * N + s * RPT, RPT), :])
        for k in range(RPT // ZCH):
            pltpu.async_copy(zbuf, acc.at[pl.ds(s * RPT + k * ZCH, ZCH), :],
                             isems[0])
        for k in range(RPT // ZCH):
            pltpu.make_async_copy(
                zbuf, acc.at[pl.ds(s * RPT + k * ZCH, ZCH), :],
                isems[0]).wait()
        plsc.subcore_barrier()

    layer(e0, e1)
    layer(e1, e2)
    layer(e2, e3)

    lanes = lax.iota(jnp.int32, 16)
    for h in range(2):
        base = c * BATCH + (s * 2 + h) * CH
        pltpu.sync_copy(usersb.at[pl.ds(base, CH)], ebuf.at[0, 0])
        pltpu.sync_copy(itemsb.at[pl.ds(base, CH)], ebuf.at[0, 1])
        for a, arr in enumerate((e0, e1, e2, e3)):
            pltpu.sync_copy(arr.at[ebuf.at[0, 0]], ub, add=(a > 0))
            pltpu.sync_copy(arr.at[ebuf.at[0, 1]], ib2, add=(a > 0))

        def dot(g, carry):
            res = jnp.zeros((16,), jnp.float32)
            for t in range(16):
                p = g * 16 + t
                prod = (ub[p, pl.ds(0, 16)] * ib2[p, pl.ds(0, 16)]
                        + ub[p, pl.ds(16, 16)] * ib2[p, pl.ds(16, 16)])
                val = jnp.sum(prod) * jnp.float32(1.0 / 16.0)
                res = jnp.where(lanes == t, val, res)
            sbuf[pl.ds(g * 16, 16)] = res
            return carry
        lax.fori_loop(0, CH // 16, dot, 0)
        pltpu.sync_copy(sbuf,
                        scores.at[pl.ds(c * BATCH + s * PPT + h * CH, CH)])


_sc_call = functools.partial(
    pl.kernel,
    out_type=[
        jax.ShapeDtypeStruct((NC * N, HALF), jnp.float32),
        jax.ShapeDtypeStruct((NC * N, HALF), jnp.float32),
        jax.ShapeDtypeStruct((NC * N, HALF), jnp.float32),
        jax.ShapeDtypeStruct((NC * BATCH,), jnp.float32),
    ],
    mesh=plsc.VectorSubcoreMesh(core_axis_name="c", subcore_axis_name="s"),
    compiler_params=pltpu.CompilerParams(use_tc_tiling_on_sc=False,
                                         needs_layout_passes=False),
    scratch_types=[
        pltpu.VMEM_SHARED((N, HALF), jnp.float32),
        pltpu.VMEM((3, 6, CH), jnp.int32),
        pltpu.VMEM((2, BLK, HALF), jnp.float32),
        pltpu.VMEM((ZCH, HALF), jnp.float32),
        pltpu.VMEM((CH, HALF), jnp.float32),
        pltpu.VMEM((CH, HALF), jnp.float32),
        pltpu.VMEM((CH,), jnp.float32),
        pltpu.SemaphoreType.DMA,
        pltpu.SemaphoreType.DMA,
        pltpu.SemaphoreType.DMA,
        pltpu.SemaphoreType.DMA,
        pltpu.SemaphoreType.DMA,
        pltpu.SemaphoreType.DMA,
        pltpu.SemaphoreType.DMA,
    ],
)(_sc_body)


def kernel(users, items, user_emb, item_emb, edge_index, edge_weight):
    row = edge_index[0]
    col = edge_index[1]
    all_emb = jnp.concatenate([user_emb, item_emb], axis=0)
    e0 = all_emb.reshape(N, NC, HALF).transpose(1, 0, 2).reshape(NC * N, HALF)
    rowp = row.reshape(NBLK, 2, CH)
    wp = lax.bitcast_convert_type(edge_weight, jnp.int32).reshape(NBLK, 2, CH)
    cores = []
    for c in range(NC):
        colp = (col + c * N).reshape(NBLK, 2, CH)
        cores.append(jnp.concatenate([colp, rowp, wp], axis=1))
    edata = jnp.concatenate(cores, axis=0)
    usersb = jnp.concatenate([users, users + N])
    itemsb = jnp.concatenate([items + N_USERS, items + N_USERS + N])
    _, _, _, partial = _sc_call(e0, edata, usersb, itemsb)
    return partial[:BATCH] + partial[BATCH:]
